# baseline (speedup 1.0000x reference)
import functools

import jax
import jax.numpy as jnp
from jax import lax
from jax.experimental import pallas as pl
from jax.experimental.pallas import tpu as pltpu
from jax.experimental.pallas import tpu_sc as plsc

EMBED_DIM = 32
BATCH = 16384

NUM_CORES = 2
NUM_SUBCORES = 16
NUM_WORKERS = NUM_CORES * NUM_SUBCORES
B_PER_W = BATCH // NUM_WORKERS

_mesh = plsc.VectorSubcoreMesh(core_axis_name="c", subcore_axis_name="s")


@functools.partial(
    pl.kernel,
    mesh=_mesh,
    out_type=jax.ShapeDtypeStruct((4, 128, 8, 128), jnp.float32),
    scratch_types=[
        pltpu.VMEM((B_PER_W,), jnp.int32),
        pltpu.VMEM((B_PER_W, EMBED_DIM), jnp.float32),
        pltpu.VMEM((4, 4, 8, 128), jnp.float32),
        pltpu.SemaphoreType.DMA,
    ],
    compiler_params=pltpu.CompilerParams(
        use_tc_tiling_on_sc=False, needs_layout_passes=False
    ),
)
def _embed_gather(idx_hbm, table_hbm, out_hbm, idx_v, rows_v, outbuf, sem):
    wid = lax.axis_index("s") * NUM_CORES + lax.axis_index("c")
    base = wid * B_PER_W
    pltpu.sync_copy(idx_hbm.at[pl.ds(base, B_PER_W)], idx_v)
    pltpu.async_copy(table_hbm.at[idx_v], rows_v, sem).wait()

    iota = lax.iota(jnp.int32, 16)
    cols = [jnp.full((16,), j, jnp.int32) for j in range(EMBED_DIM)]

    for g in range(B_PER_W // 16):
        rowsel = iota + (g * 16)
        jt = g // 8
        boff = (g % 8) * 16
        for j in range(EMBED_DIM):
            v = plsc.load_gather(rows_v, [rowsel, cols[j]])
            outbuf[j // 8, jt, j % 8, pl.ds(boff, 16)] = v

    jt0 = base // 128
    for ib in range(4):
        pltpu.sync_copy(outbuf.at[ib], out_hbm.at[ib, pl.ds(jt0, 4)])


def kernel(user_id, table):
    out4 = _embed_gather(user_id, table)
    return out4.transpose(1, 3, 0, 2).reshape(BATCH, EMBED_DIM)


# speedup vs baseline: 1.2682x; 1.2682x over previous
import functools

import jax
import jax.numpy as jnp
from jax import lax
from jax.experimental import pallas as pl
from jax.experimental.pallas import tpu as pltpu
from jax.experimental.pallas import tpu_sc as plsc

EMBED_DIM = 32
BATCH = 16384

NUM_CORES = 2
NUM_SUBCORES = 16
NUM_WORKERS = NUM_CORES * NUM_SUBCORES
B_PER_W = BATCH // NUM_WORKERS

_mesh = plsc.VectorSubcoreMesh(core_axis_name="c", subcore_axis_name="s")


@functools.partial(
    pl.kernel,
    mesh=_mesh,
    out_type=jax.ShapeDtypeStruct((4, 128, 8, 128), jnp.float32),
    scratch_types=[
        pltpu.VMEM((B_PER_W,), jnp.int32),
        pltpu.VMEM((B_PER_W, EMBED_DIM), jnp.float32),
        pltpu.VMEM((4, 4, 8, 128), jnp.float32),
        pltpu.SemaphoreType.DMA,
    ],
    compiler_params=pltpu.CompilerParams(
        use_tc_tiling_on_sc=False, needs_layout_passes=False
    ),
)
def _embed_gather(idx_hbm, table_hbm, out_hbm, idx_v, rows_v, outbuf, sem):
    wid = lax.axis_index("s") * NUM_CORES + lax.axis_index("c")
    base = wid * B_PER_W
    pltpu.sync_copy(idx_hbm.at[pl.ds(base, B_PER_W)], idx_v)
    pltpu.async_copy(table_hbm.at[idx_v], rows_v, sem).wait()

    iota = lax.iota(jnp.int32, 16)
    cols = [jnp.full((16,), j, jnp.int32) for j in range(EMBED_DIM)]

    @plsc.parallel_loop(0, B_PER_W // 16, step=1, unroll=4)
    def _transpose(g):
        rowsel = iota + g * 16
        jt = g // 8
        boff = (g % 8) * 16
        for j in range(EMBED_DIM):
            v = plsc.load_gather(rows_v, [rowsel, cols[j]])
            outbuf[j // 8, jt, j % 8, pl.ds(boff, 16)] = v

    jt0 = base // 128
    for ib in range(4):
        pltpu.sync_copy(outbuf.at[ib], out_hbm.at[ib, pl.ds(jt0, 4)])


def kernel(user_id, table):
    out4 = _embed_gather(user_id, table)
    return out4.transpose(1, 3, 0, 2).reshape(BATCH, EMBED_DIM)


# speedup vs baseline: 1.2896x; 1.0169x over previous
import functools

import jax
import jax.numpy as jnp
from jax import lax
from jax.experimental import pallas as pl
from jax.experimental.pallas import tpu as pltpu
from jax.experimental.pallas import tpu_sc as plsc

EMBED_DIM = 32
BATCH = 16384

NUM_CORES = 2
NUM_SUBCORES = 16
NUM_WORKERS = NUM_CORES * NUM_SUBCORES
B_PER_W = BATCH // NUM_WORKERS

_mesh = plsc.VectorSubcoreMesh(core_axis_name="c", subcore_axis_name="s")


@functools.partial(
    pl.kernel,
    mesh=_mesh,
    out_type=jax.ShapeDtypeStruct((4, 128, 8, 128), jnp.float32),
    scratch_types=[
        pltpu.VMEM((B_PER_W,), jnp.int32),
        pltpu.VMEM((B_PER_W, EMBED_DIM), jnp.float32),
        pltpu.VMEM((4, 4, 8, 128), jnp.float32),
        pltpu.SemaphoreType.DMA,
    ],
    compiler_params=pltpu.CompilerParams(
        use_tc_tiling_on_sc=False, needs_layout_passes=False
    ),
)
def _embed_gather(idx_hbm, table_hbm, out_hbm, idx_v, rows_v, outbuf, sem):
    wid = lax.axis_index("s") * NUM_CORES + lax.axis_index("c")
    base = wid * B_PER_W
    pltpu.sync_copy(idx_hbm.at[pl.ds(base, B_PER_W)], idx_v)
    pltpu.async_copy(table_hbm.at[idx_v], rows_v, sem).wait()

    iota = lax.iota(jnp.int32, 16)
    cols = [jnp.full((16,), j, jnp.int32) for j in range(EMBED_DIM)]

    @plsc.parallel_loop(0, B_PER_W // 16, step=1, unroll=8)
    def _transpose(g):
        rowsel = iota + g * 16
        jt = g // 8
        boff = (g % 8) * 16
        for j in range(EMBED_DIM):
            v = plsc.load_gather(rows_v, [rowsel, cols[j]])
            outbuf[j // 8, jt, j % 8, pl.ds(boff, 16)] = v

    jt0 = base // 128
    for ib in range(4):
        pltpu.sync_copy(outbuf.at[ib], out_hbm.at[ib, pl.ds(jt0, 4)])


def kernel(user_id, table):
    out4 = _embed_gather(user_id, table)
    return out4.transpose(1, 3, 0, 2).reshape(BATCH, EMBED_DIM)
